# baseline (speedup 1.0000x reference)
import functools

import jax
import jax.numpy as jnp
from jax import lax
from jax.experimental import pallas as pl
from jax.experimental.pallas import tpu as pltpu
from jax.experimental.pallas import tpu_sc as plsc

V = 100000
D = 64
C = 200
H = 128

BW = 2048
VPAD = 100352
NB = VPAD // BW

_NC = 2
_NS = 16
_NW = _NC * _NS
C_PAD = 256
ROWS_PER_W = C_PAD // _NW

@functools.cache
def _make_sc_gather():
    mesh = plsc.VectorSubcoreMesh(core_axis_name="c", subcore_axis_name="s")

    @functools.partial(
        pl.kernel,
        mesh=mesh,
        out_type=jax.ShapeDtypeStruct((C_PAD, D), jnp.float32),
        scratch_types=[
            pltpu.VMEM((ROWS_PER_W,), jnp.int32),
            pltpu.VMEM((ROWS_PER_W, D), jnp.float32),
            pltpu.SemaphoreType.DMA,
        ],
        compiler_params=pltpu.CompilerParams(use_tc_tiling_on_sc=False),
    )
    def _sc_gather(idx_hbm, table_hbm, out_hbm, idx_v, rows_v, sem):
        wid = lax.axis_index("s") * _NC + lax.axis_index("c")
        base = wid * ROWS_PER_W
        pltpu.sync_copy(idx_hbm.at[pl.ds(base, ROWS_PER_W)], idx_v)
        pltpu.async_copy(table_hbm.at[idx_v], rows_v, sem).wait()
        pltpu.sync_copy(rows_v, out_hbm.at[pl.ds(base, ROWS_PER_W)])

    return _sc_gather


def _mlp_body(emb_ref, w1_ref, b1_ref, w2_ref, b2_ref, out_ref, h_ref):
    j = pl.program_id(0)

    @pl.when(j == 0)
    def _():
        h = jnp.dot(emb_ref[...], w1_ref[...], preferred_element_type=jnp.float32)
        h_ref[...] = jnp.maximum(h + b1_ref[...], 0.0)

    out_ref[:, pl.ds(j * BW, BW)] = (
        jnp.dot(h_ref[...], w2_ref[...], preferred_element_type=jnp.float32)
        + b2_ref[...]
    )

    @pl.when(j == NB - 1)
    def _():
        x = out_ref[...]
        col = lax.broadcasted_iota(jnp.int32, (1, VPAD), 1)
        valid = col < V
        m = jnp.max(jnp.where(valid, x, -jnp.inf))
        s = jnp.sum(jnp.where(valid, jnp.exp(x - m), 0.0))
        out_ref[...] = x - (m + jnp.log(s))


_mlp_call = pl.pallas_call(
    _mlp_body,
    grid=(NB,),
    in_specs=[
        pl.BlockSpec((1, C * D), lambda j: (0, 0)),
        pl.BlockSpec((C * D, H), lambda j: (0, 0)),
        pl.BlockSpec((1, H), lambda j: (0, 0)),
        pl.BlockSpec((H, BW), lambda j: (0, j)),
        pl.BlockSpec((1, BW), lambda j: (0, j)),
    ],
    out_specs=pl.BlockSpec((1, VPAD), lambda j: (0, 0)),
    out_shape=jax.ShapeDtypeStruct((1, VPAD), jnp.float32),
    scratch_shapes=[pltpu.VMEM((1, H), jnp.float32)],
    compiler_params=pltpu.CompilerParams(
        dimension_semantics=("arbitrary",),
    ),
)


def kernel(inputs, emb, W1, b1, W2, b2):
    idx = jnp.pad(inputs.astype(jnp.int32), (0, C_PAD - C))
    rows = _make_sc_gather()(idx, emb)
    embeds = rows[:C].reshape(1, C * D)
    logits = _mlp_call(embeds, W1, b1.reshape(1, H), W2, b2.reshape(1, V))
    return logits[:, :V]


# speedup vs baseline: 1.0427x; 1.0427x over previous
import functools

import jax
import jax.numpy as jnp
from jax import lax
from jax.experimental import pallas as pl
from jax.experimental.pallas import tpu as pltpu
from jax.experimental.pallas import tpu_sc as plsc

V = 100000
D = 64
C = 200
H = 128

BW = 2048
NB = (V + BW - 1) // BW
V_FLOOR = (NB - 1) * BW
V_TAIL = V - V_FLOOR

_NC = 2
_NS = 16
_NW = _NC * _NS
ROWS_PER_W = 8
_ACTIVE_W = C // ROWS_PER_W


@functools.cache
def _make_sc_gather():
    mesh = plsc.VectorSubcoreMesh(core_axis_name="c", subcore_axis_name="s")

    @functools.partial(
        pl.kernel,
        mesh=mesh,
        out_type=jax.ShapeDtypeStruct((C, D), jnp.float32),
        scratch_types=[
            pltpu.VMEM((ROWS_PER_W,), jnp.int32),
            pltpu.VMEM((ROWS_PER_W, D), jnp.float32),
            pltpu.SemaphoreType.DMA,
        ],
        compiler_params=pltpu.CompilerParams(use_tc_tiling_on_sc=False),
    )
    def _sc_gather(idx_hbm, table_hbm, out_hbm, idx_v, rows_v, sem):
        wid = lax.axis_index("s") * _NC + lax.axis_index("c")

        @pl.when(wid < _ACTIVE_W)
        def _():
            base = wid * ROWS_PER_W
            pltpu.sync_copy(idx_hbm.at[pl.ds(base, ROWS_PER_W)], idx_v)
            pltpu.async_copy(table_hbm.at[idx_v], rows_v, sem).wait()
            pltpu.sync_copy(rows_v, out_hbm.at[pl.ds(base, ROWS_PER_W)])

    return _sc_gather


def _mlp_body(emb_ref, w1_ref, b1_ref, w2_ref, b2_ref, out_ref, h_ref):
    j = pl.program_id(0)

    @pl.when(j == 0)
    def _():
        h = jnp.dot(emb_ref[...], w1_ref[...], preferred_element_type=jnp.float32)
        h_ref[...] = jnp.maximum(h + b1_ref[...], 0.0)

    logits = (
        jnp.dot(h_ref[...], w2_ref[...], preferred_element_type=jnp.float32)
        + b2_ref[...]
    )

    @pl.when(j < NB - 1)
    def _():
        out_ref[:, pl.ds(j * BW, BW)] = logits

    @pl.when(j == NB - 1)
    def _():
        out_ref[:, V_FLOOR:V] = logits[:, :V_TAIL]
        x = out_ref[...]
        m = jnp.max(x)
        s = jnp.sum(jnp.exp(x - m))
        out_ref[...] = x - (m + jnp.log(s))


_mlp_call = pl.pallas_call(
    _mlp_body,
    grid=(NB,),
    in_specs=[
        pl.BlockSpec((1, C * D), lambda j: (0, 0)),
        pl.BlockSpec((C * D, H), lambda j: (0, 0)),
        pl.BlockSpec((1, H), lambda j: (0, 0)),
        pl.BlockSpec((H, BW), lambda j: (0, j)),
        pl.BlockSpec((1, BW), lambda j: (0, j)),
    ],
    out_specs=pl.BlockSpec((1, V), lambda j: (0, 0)),
    out_shape=jax.ShapeDtypeStruct((1, V), jnp.float32),
    scratch_shapes=[pltpu.VMEM((1, H), jnp.float32)],
    compiler_params=pltpu.CompilerParams(
        dimension_semantics=("arbitrary",),
    ),
)


def kernel(inputs, emb, W1, b1, W2, b2):
    rows = _make_sc_gather()(inputs.astype(jnp.int32), emb)
    embeds = rows.reshape(1, C * D)
    return _mlp_call(embeds, W1, b1.reshape(1, H), W2, b2.reshape(1, V))


# speedup vs baseline: 1.3242x; 1.2699x over previous
import jax
import jax.numpy as jnp
from jax.experimental import pallas as pl
from jax.experimental.pallas import tpu as pltpu

V = 100000
D = 64
C = 200
H = 128

BW = 2048
NB = (V + BW - 1) // BW
V_FLOOR = (NB - 1) * BW
V_TAIL = V - V_FLOOR


def _gather_body(idx_ref, emb_hbm, out_ref, sem):
    copies = []
    for i in range(C):
        cp = pltpu.make_async_copy(
            emb_hbm.at[pl.ds(idx_ref[i], 1), :],
            out_ref.at[pl.ds(i, 1), :],
            sem,
        )
        cp.start()
        copies.append(cp)
    for cp in copies:
        cp.wait()


_gather_call = pl.pallas_call(
    _gather_body,
    grid_spec=pltpu.PrefetchScalarGridSpec(
        num_scalar_prefetch=1,
        grid=(1,),
        in_specs=[pl.BlockSpec(memory_space=pl.ANY)],
        out_specs=pl.BlockSpec((C, D), lambda j, idx: (0, 0)),
        scratch_shapes=[pltpu.SemaphoreType.DMA],
    ),
    out_shape=jax.ShapeDtypeStruct((C, D), jnp.float32),
)


def _mlp_body(emb_ref, w1_ref, b1_ref, w2_ref, b2_ref, out_ref, h_ref):
    j = pl.program_id(0)

    @pl.when(j == 0)
    def _():
        h = jnp.dot(emb_ref[...], w1_ref[...], preferred_element_type=jnp.float32)
        h_ref[...] = jnp.maximum(h + b1_ref[...], 0.0)

    logits = (
        jnp.dot(h_ref[...], w2_ref[...], preferred_element_type=jnp.float32)
        + b2_ref[...]
    )

    @pl.when(j < NB - 1)
    def _():
        out_ref[:, pl.ds(j * BW, BW)] = logits

    @pl.when(j == NB - 1)
    def _():
        out_ref[:, V_FLOOR:V] = logits[:, :V_TAIL]
        x = out_ref[...]
        m = jnp.max(x)
        s = jnp.sum(jnp.exp(x - m))
        out_ref[...] = x - (m + jnp.log(s))


_mlp_call = pl.pallas_call(
    _mlp_body,
    grid=(NB,),
    in_specs=[
        pl.BlockSpec((1, C * D), lambda j: (0, 0)),
        pl.BlockSpec((C * D, H), lambda j: (0, 0)),
        pl.BlockSpec((1, H), lambda j: (0, 0)),
        pl.BlockSpec((H, BW), lambda j: (0, j)),
        pl.BlockSpec((1, BW), lambda j: (0, j)),
    ],
    out_specs=pl.BlockSpec((1, V), lambda j: (0, 0)),
    out_shape=jax.ShapeDtypeStruct((1, V), jnp.float32),
    scratch_shapes=[pltpu.VMEM((1, H), jnp.float32)],
    compiler_params=pltpu.CompilerParams(
        dimension_semantics=("arbitrary",),
    ),
)


def kernel(inputs, emb, W1, b1, W2, b2):
    rows = _gather_call(inputs.astype(jnp.int32), emb)
    embeds = rows.reshape(1, C * D)
    return _mlp_call(embeds, W1, b1.reshape(1, H), W2, b2.reshape(1, V))


# speedup vs baseline: 1.6979x; 1.2822x over previous
import jax
import jax.numpy as jnp
from jax import lax
from jax.experimental import pallas as pl
from jax.experimental.pallas import tpu as pltpu

V = 100000
D = 64
C = 200
H = 128

BW = 2048
NB = (V + BW - 1) // BW
V_FLOOR = (NB - 1) * BW
V_TAIL = V - V_FLOOR


def _one_hot_row(idx_scalar, blk):
    lane = lax.broadcasted_iota(jnp.int32, (1, 128), 1)
    onehot = (lane == (idx_scalar % 128)).astype(jnp.float32)
    return lax.dot_general(
        onehot, blk,
        dimension_numbers=(((1,), (1,)), ((), ())),
        preferred_element_type=jnp.float32,
    )


def _gather_body(idx_ref, blk_a_ref, blk_b_ref, out_ref):
    t = pl.program_id(0)
    row_a = _one_hot_row(idx_ref[2 * t], blk_a_ref[...])
    row_b = _one_hot_row(idx_ref[2 * t + 1], blk_b_ref[...])
    out_ref[...] = jnp.concatenate([row_a, row_b], axis=1)


_gather_call = pl.pallas_call(
    _gather_body,
    grid_spec=pltpu.PrefetchScalarGridSpec(
        num_scalar_prefetch=1,
        grid=(C // 2,),
        in_specs=[
            pl.BlockSpec((D, 128), lambda t, idx: (0, idx[2 * t] // 128)),
            pl.BlockSpec((D, 128), lambda t, idx: (0, idx[2 * t + 1] // 128)),
        ],
        out_specs=pl.BlockSpec((1, 2 * D), lambda t, idx: (0, t)),
    ),
    out_shape=jax.ShapeDtypeStruct((1, C * D), jnp.float32),
)


def _mlp_body(emb_ref, w1_ref, b1_ref, w2t_ref, b2_ref, out_ref, h_ref):
    j = pl.program_id(0)

    @pl.when(j == 0)
    def _():
        h = jnp.dot(emb_ref[...], w1_ref[...], preferred_element_type=jnp.float32)
        h_ref[...] = jnp.maximum(h + b1_ref[...], 0.0)

    logits = (
        lax.dot_general(
            h_ref[...], w2t_ref[...],
            dimension_numbers=(((1,), (1,)), ((), ())),
            preferred_element_type=jnp.float32,
        )
        + b2_ref[...]
    )

    @pl.when(j < NB - 1)
    def _():
        out_ref[:, pl.ds(j * BW, BW)] = logits

    @pl.when(j == NB - 1)
    def _():
        out_ref[:, V_FLOOR:V] = logits[:, :V_TAIL]
        x = out_ref[...]
        m = jnp.max(x)
        s = jnp.sum(jnp.exp(x - m))
        out_ref[...] = x - (m + jnp.log(s))


_mlp_call = pl.pallas_call(
    _mlp_body,
    grid=(NB,),
    in_specs=[
        pl.BlockSpec((1, C * D), lambda j: (0, 0)),
        pl.BlockSpec((C * D, H), lambda j: (0, 0)),
        pl.BlockSpec((1, H), lambda j: (0, 0)),
        pl.BlockSpec((BW, H), lambda j: (j, 0)),
        pl.BlockSpec((1, BW), lambda j: (0, j)),
    ],
    out_specs=pl.BlockSpec((1, V), lambda j: (0, 0)),
    out_shape=jax.ShapeDtypeStruct((1, V), jnp.float32),
    scratch_shapes=[pltpu.VMEM((1, H), jnp.float32)],
    compiler_params=pltpu.CompilerParams(
        dimension_semantics=("arbitrary",),
    ),
)


def kernel(inputs, emb, W1, b1, W2, b2):
    embT = emb.T
    idx = inputs.astype(jnp.int32)
    embeds = _gather_call(idx, embT, embT)
    return _mlp_call(embeds, W1, b1.reshape(1, H), W2.T,
                     b2.reshape(1, V))


# speedup vs baseline: 3.4090x; 2.0078x over previous
import jax
import jax.numpy as jnp
from jax import lax
from jax.experimental import pallas as pl
from jax.experimental.pallas import tpu as pltpu

V = 100000
D = 64
C = 200
H = 128

BW = 4096
NB = (V + BW - 1) // BW
V_FLOOR = (NB - 1) * BW
V_TAIL = V - V_FLOOR


def _one_hot_row(idx_scalar, blk):
    lane = lax.broadcasted_iota(jnp.int32, (1, 128), 1)
    onehot = (lane == (idx_scalar % 128)).astype(jnp.float32)
    return lax.dot_general(
        onehot, blk,
        dimension_numbers=(((1,), (1,)), ((), ())),
        preferred_element_type=jnp.float32,
    )


_GPER = 8


def _gather_body(idx_ref, *refs):
    t = pl.program_id(0)
    blk_refs, out_ref = refs[:_GPER], refs[_GPER]
    rows = [
        _one_hot_row(idx_ref[_GPER * t + k], blk_refs[k][...])
        for k in range(_GPER)
    ]
    out_ref[...] = jnp.concatenate(rows, axis=1)


def _gather_spec(k):
    return pl.BlockSpec((D, 128), lambda t, idx: (0, idx[_GPER * t + k] // 128))


_gather_call = pl.pallas_call(
    _gather_body,
    grid_spec=pltpu.PrefetchScalarGridSpec(
        num_scalar_prefetch=1,
        grid=(C // _GPER,),
        in_specs=[_gather_spec(k) for k in range(_GPER)],
        out_specs=pl.BlockSpec((1, _GPER * D), lambda t, idx: (0, t)),
    ),
    out_shape=jax.ShapeDtypeStruct((1, C * D), jnp.float32),
    compiler_params=pltpu.CompilerParams(
        dimension_semantics=("parallel",),
    ),
)


def _mlp_body(emb_ref, w1_ref, b1_ref, w2t_ref, b2_ref, out_ref, h_ref):
    j = pl.program_id(0)

    @pl.when(j == 0)
    def _():
        h = jnp.dot(emb_ref[...], w1_ref[...], preferred_element_type=jnp.float32)
        h_ref[...] = jnp.maximum(h + b1_ref[...], 0.0)

    logits = (
        lax.dot_general(
            h_ref[...], w2t_ref[...],
            dimension_numbers=(((1,), (1,)), ((), ())),
            preferred_element_type=jnp.float32,
        )
        + b2_ref[...]
    )

    @pl.when(j < NB - 1)
    def _():
        out_ref[:, pl.ds(j * BW, BW)] = logits

    @pl.when(j == NB - 1)
    def _():
        out_ref[:, V_FLOOR:V] = logits[:, :V_TAIL]
        x = out_ref[...]
        m = jnp.max(x)
        s = jnp.sum(jnp.exp(x - m))
        out_ref[...] = x - (m + jnp.log(s))


_mlp_call = pl.pallas_call(
    _mlp_body,
    grid=(NB,),
    in_specs=[
        pl.BlockSpec((1, C * D), lambda j: (0, 0)),
        pl.BlockSpec((C * D, H), lambda j: (0, 0)),
        pl.BlockSpec((1, H), lambda j: (0, 0)),
        pl.BlockSpec((BW, H), lambda j: (j, 0)),
        pl.BlockSpec((1, BW), lambda j: (0, j)),
    ],
    out_specs=pl.BlockSpec((1, V), lambda j: (0, 0)),
    out_shape=jax.ShapeDtypeStruct((1, V), jnp.float32),
    scratch_shapes=[pltpu.VMEM((1, H), jnp.float32)],
    compiler_params=pltpu.CompilerParams(
        dimension_semantics=("arbitrary",),
    ),
)


def kernel(inputs, emb, W1, b1, W2, b2):
    embT = emb.T
    idx = inputs.astype(jnp.int32)
    embeds = _gather_call(idx, *([embT] * _GPER))
    return _mlp_call(embeds, W1, b1.reshape(1, H), W2.T,
                     b2.reshape(1, V))


# speedup vs baseline: 3.9871x; 1.1696x over previous
import jax
import jax.numpy as jnp
from jax import lax
from jax.experimental import pallas as pl
from jax.experimental.pallas import tpu as pltpu

V = 100000
D = 64
C = 200
H = 128

BW = 4096
NB = (V + BW - 1) // BW
V_FLOOR = (NB - 1) * BW
V_TAIL = V - V_FLOOR


def _one_hot_row(idx_scalar, blk):
    lane = lax.broadcasted_iota(jnp.int32, (1, 128), 1)
    onehot = (lane == (idx_scalar % 128)).astype(jnp.float32)
    return lax.dot_general(
        onehot, blk,
        dimension_numbers=(((1,), (1,)), ((), ())),
        preferred_element_type=jnp.float32,
    )


_GPER = 40


def _gather_body(idx_ref, *refs):
    t = pl.program_id(0)
    blk_refs, out_ref = refs[:_GPER], refs[_GPER]
    rows = [
        _one_hot_row(idx_ref[_GPER * t + k], blk_refs[k][...])
        for k in range(_GPER)
    ]
    out_ref[...] = jnp.concatenate(rows, axis=1)


def _gather_spec(k):
    return pl.BlockSpec((D, 128), lambda t, idx: (0, idx[_GPER * t + k] // 128))


_gather_call = pl.pallas_call(
    _gather_body,
    grid_spec=pltpu.PrefetchScalarGridSpec(
        num_scalar_prefetch=1,
        grid=(C // _GPER,),
        in_specs=[_gather_spec(k) for k in range(_GPER)],
        out_specs=pl.BlockSpec((1, _GPER * D), lambda t, idx: (0, t)),
    ),
    out_shape=jax.ShapeDtypeStruct((1, C * D), jnp.float32),
    compiler_params=pltpu.CompilerParams(
        dimension_semantics=("parallel",),
    ),
)


def _mlp_body(emb_ref, w1_ref, b1_ref, w2t_ref, b2_ref, out_ref, h_ref):
    j = pl.program_id(0)

    @pl.when(j == 0)
    def _():
        h = jnp.dot(emb_ref[...], w1_ref[...], preferred_element_type=jnp.float32)
        h_ref[...] = jnp.maximum(h + b1_ref[...], 0.0)

    logits = (
        lax.dot_general(
            h_ref[...], w2t_ref[...],
            dimension_numbers=(((1,), (1,)), ((), ())),
            preferred_element_type=jnp.float32,
        )
        + b2_ref[...]
    )

    @pl.when(j < NB - 1)
    def _():
        out_ref[:, pl.ds(pl.multiple_of(j * BW, BW), BW)] = logits

    @pl.when(j == NB - 1)
    def _():
        out_ref[:, V_FLOOR:V] = logits[:, :V_TAIL]
        x = out_ref[...]
        m = jnp.max(x)
        s = jnp.sum(jnp.exp(x - m))
        out_ref[...] = x - (m + jnp.log(s))


_mlp_call = pl.pallas_call(
    _mlp_body,
    grid=(NB,),
    in_specs=[
        pl.BlockSpec((1, C * D), lambda j: (0, 0)),
        pl.BlockSpec((C * D, H), lambda j: (0, 0)),
        pl.BlockSpec((1, H), lambda j: (0, 0)),
        pl.BlockSpec((BW, H), lambda j: (j, 0)),
        pl.BlockSpec((1, BW), lambda j: (0, j)),
    ],
    out_specs=pl.BlockSpec((1, V), lambda j: (0, 0)),
    out_shape=jax.ShapeDtypeStruct((1, V), jnp.float32),
    scratch_shapes=[pltpu.VMEM((1, H), jnp.float32)],
    compiler_params=pltpu.CompilerParams(
        dimension_semantics=("arbitrary",),
    ),
)


def kernel(inputs, emb, W1, b1, W2, b2):
    embT = emb.T
    idx = inputs.astype(jnp.int32)
    embeds = _gather_call(idx, *([embT] * _GPER))
    return _mlp_call(embeds, W1, b1.reshape(1, H), W2.T,
                     b2.reshape(1, V))


# speedup vs baseline: 4.4816x; 1.1240x over previous
import jax
import jax.numpy as jnp
from jax import lax
from jax.experimental import pallas as pl
from jax.experimental.pallas import tpu as pltpu

V = 100000
D = 64
C = 200
H = 128

BW = 8192
NB = (V + BW - 1) // BW
V_FLOOR = (NB - 1) * BW
V_TAIL = V - V_FLOOR


def _one_hot_row(idx_scalar, blk):
    lane = lax.broadcasted_iota(jnp.int32, (1, 128), 1)
    onehot = (lane == (idx_scalar % 128)).astype(jnp.float32)
    return lax.dot_general(
        onehot, blk,
        dimension_numbers=(((1,), (1,)), ((), ())),
        preferred_element_type=jnp.float32,
    )


_GPER = 100


def _gather_body(idx_ref, *refs):
    t = pl.program_id(0)
    blk_refs, out_ref = refs[:_GPER], refs[_GPER]
    rows = [
        _one_hot_row(idx_ref[_GPER * t + k], blk_refs[k][...])
        for k in range(_GPER)
    ]
    out_ref[...] = jnp.concatenate(rows, axis=1)


def _gather_spec(k):
    return pl.BlockSpec((D, 128), lambda t, idx: (0, idx[_GPER * t + k] // 128))


_gather_call = pl.pallas_call(
    _gather_body,
    grid_spec=pltpu.PrefetchScalarGridSpec(
        num_scalar_prefetch=1,
        grid=(C // _GPER,),
        in_specs=[_gather_spec(k) for k in range(_GPER)],
        out_specs=pl.BlockSpec((1, _GPER * D), lambda t, idx: (0, t)),
    ),
    out_shape=jax.ShapeDtypeStruct((1, C * D), jnp.float32),
    compiler_params=pltpu.CompilerParams(
        dimension_semantics=("parallel",),
    ),
)


def _mlp_body(emb_ref, w1_ref, b1_ref, w2t_ref, b2_ref, out_ref, h_ref):
    j = pl.program_id(0)

    @pl.when(j == 0)
    def _():
        h = jnp.dot(emb_ref[...], w1_ref[...], preferred_element_type=jnp.float32)
        h_ref[...] = jnp.maximum(h + b1_ref[...], 0.0)

    logits = (
        lax.dot_general(
            h_ref[...], w2t_ref[...],
            dimension_numbers=(((1,), (1,)), ((), ())),
            preferred_element_type=jnp.float32,
        )
        + b2_ref[...]
    )

    @pl.when(j < NB - 1)
    def _():
        out_ref[:, pl.ds(pl.multiple_of(j * BW, BW), BW)] = logits

    @pl.when(j == NB - 1)
    def _():
        out_ref[:, V_FLOOR:V] = logits[:, :V_TAIL]
        x = out_ref[...]
        m = jnp.max(x)
        s = jnp.sum(jnp.exp(x - m))
        out_ref[...] = x - (m + jnp.log(s))


_mlp_call = pl.pallas_call(
    _mlp_body,
    grid=(NB,),
    in_specs=[
        pl.BlockSpec((1, C * D), lambda j: (0, 0)),
        pl.BlockSpec((C * D, H), lambda j: (0, 0)),
        pl.BlockSpec((1, H), lambda j: (0, 0)),
        pl.BlockSpec((BW, H), lambda j: (j, 0)),
        pl.BlockSpec((1, BW), lambda j: (0, j)),
    ],
    out_specs=pl.BlockSpec((1, V), lambda j: (0, 0)),
    out_shape=jax.ShapeDtypeStruct((1, V), jnp.float32),
    scratch_shapes=[pltpu.VMEM((1, H), jnp.float32)],
    compiler_params=pltpu.CompilerParams(
        dimension_semantics=("arbitrary",),
    ),
)


def kernel(inputs, emb, W1, b1, W2, b2):
    embT = emb.T
    idx = inputs.astype(jnp.int32)
    embeds = _gather_call(idx, *([embT] * _GPER))
    return _mlp_call(embeds, W1, b1.reshape(1, H), W2.T,
                     b2.reshape(1, V))


# speedup vs baseline: 5.3599x; 1.1960x over previous
import jax
import jax.numpy as jnp
from jax import lax
from jax.experimental import pallas as pl
from jax.experimental.pallas import tpu as pltpu

V = 100000
D = 64
C = 200
H = 128

BW = 8192
NB = (V + BW - 1) // BW
V_FLOOR = (NB - 1) * BW
V_TAIL = V - V_FLOOR


def _one_hot_row(idx_scalar, blk):
    lane = lax.broadcasted_iota(jnp.int32, (1, 128), 1)
    onehot = (lane == (idx_scalar % 128)).astype(jnp.float32)
    return lax.dot_general(
        onehot, blk,
        dimension_numbers=(((1,), (1,)), ((), ())),
        preferred_element_type=jnp.float32,
    )


_GPER = 200


def _gather_body(idx_ref, *refs):
    t = pl.program_id(0)
    blk_refs, out_ref = refs[:_GPER], refs[_GPER]
    rows = [
        _one_hot_row(idx_ref[_GPER * t + k], blk_refs[k][...])
        for k in range(_GPER)
    ]
    out_ref[...] = jnp.concatenate(rows, axis=1)


def _gather_spec(k):
    return pl.BlockSpec((D, 128), lambda t, idx: (0, idx[_GPER * t + k] // 128))


_gather_call = pl.pallas_call(
    _gather_body,
    grid_spec=pltpu.PrefetchScalarGridSpec(
        num_scalar_prefetch=1,
        grid=(C // _GPER,),
        in_specs=[_gather_spec(k) for k in range(_GPER)],
        out_specs=pl.BlockSpec((1, _GPER * D), lambda t, idx: (0, t)),
    ),
    out_shape=jax.ShapeDtypeStruct((1, C * D), jnp.float32),
    compiler_params=pltpu.CompilerParams(
        dimension_semantics=("parallel",),
    ),
)


def _mlp_body(emb_ref, w1_ref, b1_ref, w2t_ref, b2_ref, out_ref, h_ref):
    j = pl.program_id(0)

    @pl.when(j == 0)
    def _():
        h = jnp.dot(emb_ref[...], w1_ref[...], preferred_element_type=jnp.float32)
        h_ref[...] = jnp.maximum(h + b1_ref[...], 0.0)

    logits = (
        lax.dot_general(
            h_ref[...], w2t_ref[...],
            dimension_numbers=(((1,), (1,)), ((), ())),
            preferred_element_type=jnp.float32,
        )
        + b2_ref[...]
    )

    @pl.when(j < NB - 1)
    def _():
        out_ref[:, pl.ds(pl.multiple_of(j * BW, BW), BW)] = logits

    @pl.when(j == NB - 1)
    def _():
        out_ref[:, V_FLOOR:V] = logits[:, :V_TAIL]
        x = out_ref[...]
        m = jnp.max(x)
        s = jnp.sum(jnp.exp(x - m))
        out_ref[...] = x - (m + jnp.log(s))


_mlp_call = pl.pallas_call(
    _mlp_body,
    grid=(NB,),
    in_specs=[
        pl.BlockSpec((1, C * D), lambda j: (0, 0)),
        pl.BlockSpec((C * D, H), lambda j: (0, 0)),
        pl.BlockSpec((1, H), lambda j: (0, 0)),
        pl.BlockSpec((BW, H), lambda j: (j, 0)),
        pl.BlockSpec((1, BW), lambda j: (0, j)),
    ],
    out_specs=pl.BlockSpec((1, V), lambda j: (0, 0)),
    out_shape=jax.ShapeDtypeStruct((1, V), jnp.float32),
    scratch_shapes=[pltpu.VMEM((1, H), jnp.float32)],
    compiler_params=pltpu.CompilerParams(
        dimension_semantics=("arbitrary",),
    ),
)


def kernel(inputs, emb, W1, b1, W2, b2):
    embT = emb.T
    idx = inputs.astype(jnp.int32)
    embeds = _gather_call(idx, *([embT] * _GPER))
    return _mlp_call(embeds, W1, b1.reshape(1, H), W2.T,
                     b2.reshape(1, V))
